# baseline (speedup 1.0000x reference)
import functools

import numpy as np
import jax
import jax.numpy as jnp
from jax import lax
from jax.experimental import pallas as pl
from jax.experimental.pallas import tpu as pltpu
from jax.experimental.pallas import tpu_sc as plsc

PATCH_BANK = [128, 256, 384, 512]
CS = 128
FOV_SCALE = 32
N_LOC = 8
H = W = 1024
B, C, K = 4, 3, 4
HL = H // FOV_SCALE


def _resize_mat(p, out):
    scale = out / p
    sample_f = (np.arange(out, dtype=np.float64) + 0.5) / scale - 0.5
    x = np.abs(sample_f[:, None] - np.arange(p, dtype=np.float64)[None, :]) * min(scale, 1.0)
    w = np.maximum(0.0, 1.0 - x)
    w = w / w.sum(axis=1, keepdims=True)
    return w.astype(np.float32)


_R = [_resize_mat(p, CS) for p in PATCH_BANK]
_D = _resize_mat(H, HL)


def _router_kernel(x_ref, d_ref, dt_ref, wf_ref, bf_ref, xi_ref, yi_ref,
                   f_ref, wloc_ref, x16_ref):
    x = x_ref[0]
    xlr = []
    for c in range(C):
        xb = x[c].astype(jnp.bfloat16)
        x16_ref[0, c] = xb
        a = jnp.dot(xb, dt_ref[...], preferred_element_type=jnp.float32)
        xlr.append(jnp.dot(d_ref[...], a.astype(jnp.bfloat16),
                           preferred_element_type=jnp.float32))
    pads = []
    zr = jnp.zeros((1, HL), jnp.float32)
    zc = jnp.zeros((HL + 2, 1), jnp.float32)
    for c in range(C):
        t = jnp.concatenate([zr, xlr[c], zr], axis=0)
        pads.append(jnp.concatenate([zc, t, zc], axis=1))
    logits = []
    for k in range(K):
        acc = jnp.full((HL, HL), 0.0, jnp.float32) + bf_ref[k, 0]
        for c in range(C):
            for dy in range(3):
                for dx in range(3):
                    acc = acc + wf_ref[k, c * 9 + dy * 3 + dx] * pads[c][dy:dy + HL, dx:dx + HL]
        logits.append(acc)
    m = jnp.maximum(jnp.maximum(logits[0], logits[1]),
                    jnp.maximum(logits[2], logits[3]))
    es = [jnp.exp(l - m) for l in logits]
    tot = es[0] + es[1] + es[2] + es[3]
    fs = [e / tot for e in es]
    for k in range(K):
        f_ref[0, k] = fs[k]
    ox = (jax.lax.broadcasted_iota(jnp.int32, (N_LOC, HL), 1) == xi_ref[...]).astype(jnp.float32)
    oy = (jax.lax.broadcasted_iota(jnp.int32, (N_LOC, HL), 1) == yi_ref[...]).astype(jnp.float32)
    cols = []
    for k in range(K):
        mk = jnp.dot(ox, fs[k], preferred_element_type=jnp.float32)
        cols.append(jnp.sum(mk * oy, axis=1, keepdims=True))
    wloc_ref[0] = jnp.concatenate(cols, axis=1)


WIN = [p + CS for p in PATCH_BANK]


_SC_NC, _SC_NS = 2, 16
_SC_NW = _SC_NC * _SC_NS
_LROWS = B * N_LOC * CS * (CS // 32)
_LPW = _LROWS // _SC_NW


def _label_sc_kernel(y32_hbm, idx_hbm, rl_hbm, out_hbm,
                     idx_v, buf2_v, out_v, rl_v, sem):
    wid = lax.axis_index("s") * _SC_NC + lax.axis_index("c")
    pltpu.sync_copy(idx_hbm.at[pl.ds(wid * 2 * CS, 2 * CS)], idx_v)
    pltpu.sync_copy(rl_hbm.at[wid], rl_v)
    pltpu.async_copy(y32_hbm.at[idx_v], buf2_v, sem).wait()
    masks = [rl_v[...] == 32 * q for q in range(4)]

    @pl.loop(0, CS)
    def _(r):
        for j in range(CS // 16):
            cands = []
            for q in range(4):
                col = 16 * j + 32 * q
                if col < 128:
                    cands.append(buf2_v[2 * r, pl.ds(col, 16)])
                else:
                    cands.append(buf2_v[2 * r + 1, pl.ds(col - 128, 16)])
            g = jnp.where(masks[0], cands[0],
                          jnp.where(masks[1], cands[1],
                                    jnp.where(masks[2], cands[2], cands[3])))
            out_v[r, pl.ds(16 * j, 16)] = g

    pltpu.sync_copy(out_v, out_hbm.at[pl.ds(wid * CS, CS)])


def _gather_labels(Y, xls, yls):
    y32 = Y.reshape(-1, 128)
    r = jnp.arange(CS, dtype=jnp.int32)
    b = jnp.arange(B, dtype=jnp.int32)
    cb = yls // 128
    rl = yls - cb * 128
    rowa = (b[None, :, None] * (H * W // 128)
            + (xls[:, None, None] + r[None, None, :]) * (W // 128)
            + cb[:, None, None])
    rowb = jnp.minimum(rowa + 1, H * W * B // 128 - 1)
    gidx = jnp.stack([rowa, rowb], axis=-1).astype(jnp.int32).reshape(-1)
    rl16 = jnp.broadcast_to(jnp.repeat(rl, B)[:, None].astype(jnp.int32),
                            (_SC_NW, 16))
    mesh = plsc.VectorSubcoreMesh(core_axis_name="c", subcore_axis_name="s")
    out = pl.kernel(
        _label_sc_kernel,
        out_type=jax.ShapeDtypeStruct((B * N_LOC * CS, CS), jnp.int32),
        mesh=mesh,
        scratch_types=[
            pltpu.VMEM((2 * CS,), jnp.int32),
            pltpu.VMEM((2 * CS, CS), jnp.int32),
            pltpu.VMEM((CS, CS), jnp.int32),
            pltpu.VMEM((16,), jnp.int32),
            pltpu.SemaphoreType.DMA,
        ],
    )(y32, gidx, rl16)
    return out.reshape(B * N_LOC, CS, CS)


def _patch_kernel(idx_ref, x_ref, wloc_ref,
                  g0_ref, g1_ref, g2_ref, g3_ref,
                  r0_ref, r1_ref, r2_ref, r3_ref,
                  out_w_ref):
    i = pl.program_id(2)
    rrefs = [r0_ref, r1_ref, r2_ref, r3_ref]
    grefs = [g0_ref, g1_ref, g2_ref, g3_ref]
    acc = jnp.zeros((CS, CS), jnp.float32)
    for k in range(K):
        p = PATCH_BANK[k]
        dx = pl.multiple_of(idx_ref[i, k], 32)
        base = pl.multiple_of(idx_ref[i, K + k], 128)
        srow = x_ref[0, 0, pl.ds(dx, p), pl.ds(base, WIN[k])]
        t = jnp.dot(rrefs[k][...], srow,
                    preferred_element_type=jnp.float32)
        outk = jnp.dot(t.astype(jnp.bfloat16), grefs[k][0],
                       preferred_element_type=jnp.float32)
        acc = acc + wloc_ref[0, 0, 0, k] * outk
    out_w_ref[0, 0] = acc


def kernel(X, Y, xi, yi, W_fov, b_fov):
    d = jnp.asarray(_D).astype(jnp.bfloat16)
    dt = jnp.asarray(_D.T).astype(jnp.bfloat16)
    wf = W_fov.reshape(K, C * 9)
    bf = b_fov.reshape(K, 1)
    xi2 = xi.reshape(N_LOC, 1)
    yi2 = yi.reshape(N_LOC, 1)

    router_outs = pl.pallas_call(
        _router_kernel,
        grid=(B,),
        in_specs=[
            pl.BlockSpec((1, C, H, W), lambda b: (b, 0, 0, 0)),
            pl.BlockSpec((HL, H), lambda b: (0, 0)),
            pl.BlockSpec((H, HL), lambda b: (0, 0)),
            pl.BlockSpec((K, C * 9), lambda b: (0, 0)),
            pl.BlockSpec((K, 1), lambda b: (0, 0)),
            pl.BlockSpec((N_LOC, 1), lambda b: (0, 0)),
            pl.BlockSpec((N_LOC, 1), lambda b: (0, 0)),
        ],
        out_specs=[
            pl.BlockSpec((1, K, HL, HL), lambda b: (b, 0, 0, 0)),
            pl.BlockSpec((1, N_LOC, K), lambda b: (b, 0, 0)),
            pl.BlockSpec((1, C, H, W), lambda b: (b, 0, 0, 0)),
        ],
        out_shape=[
            jax.ShapeDtypeStruct((B, K, HL, HL), jnp.float32),
            jax.ShapeDtypeStruct((B, N_LOC, K), jnp.float32),
            jax.ShapeDtypeStruct((B, C, H, W), jnp.bfloat16),
        ],
    )(X, d, dt, wf, bf, xi2, yi2)
    f_xlr, wloc_bk, x16 = router_outs

    cx = xi.astype(jnp.int32) * FOV_SCALE
    cy = yi.astype(jnp.int32) * FOV_SCALE
    xls = jnp.clip(cx - CS // 2, 0, H - CS)
    yls = jnp.clip(cy - CS // 2, 0, W - CS)
    dxs = [jnp.clip(cx - p // 2, 0, H - p) for p in PATCH_BANK]
    dys = [jnp.clip(cy - p // 2, 0, W - p) for p in PATCH_BANK]
    bases = [jnp.minimum((dys[k] // 128) * 128, W - WIN[k]) for k in range(K)]
    offs = [dys[k] - bases[k] for k in range(K)]
    idx = jnp.stack(dxs + bases, axis=1)

    gcts = []
    for k in range(K):
        rkt = jnp.asarray(_R[k].T)
        zero = jnp.zeros((WIN[k], CS), jnp.float32)
        gcts.append(jax.vmap(
            lambda off, r=rkt, z=zero: jax.lax.dynamic_update_slice(z, r, (off, 0))
        )(offs[k]).astype(jnp.bfloat16))

    out_l = _gather_labels(Y, xls, yls)

    grid_spec = pltpu.PrefetchScalarGridSpec(
        num_scalar_prefetch=1,
        grid=(B, C, N_LOC),
        in_specs=[
            pl.BlockSpec((1, 1, H, W), lambda b, c, i, idx: (b, c, 0, 0)),
            pl.BlockSpec((1, 1, 1, K), lambda b, c, i, idx: (b, i, 0, 0)),
            pl.BlockSpec((1, WIN[0], CS), lambda b, c, i, idx: (i, 0, 0)),
            pl.BlockSpec((1, WIN[1], CS), lambda b, c, i, idx: (i, 0, 0)),
            pl.BlockSpec((1, WIN[2], CS), lambda b, c, i, idx: (i, 0, 0)),
            pl.BlockSpec((1, WIN[3], CS), lambda b, c, i, idx: (i, 0, 0)),
            pl.BlockSpec((CS, 128), lambda b, c, i, idx: (0, 0)),
            pl.BlockSpec((CS, 256), lambda b, c, i, idx: (0, 0)),
            pl.BlockSpec((CS, 384), lambda b, c, i, idx: (0, 0)),
            pl.BlockSpec((CS, 512), lambda b, c, i, idx: (0, 0)),
        ],
        out_specs=[
            pl.BlockSpec((1, 1, CS, CS), lambda b, c, i, idx: (i * B + b, c, 0, 0)),
        ],
    )

    out_w, = pl.pallas_call(
        _patch_kernel,
        grid_spec=grid_spec,
        out_shape=[
            jax.ShapeDtypeStruct((N_LOC * B, C, CS, CS), jnp.float32),
        ],
    )(idx, x16, wloc_bk.reshape(B, N_LOC, 1, K), gcts[0], gcts[1], gcts[2], gcts[3],
      jnp.asarray(_R[0]).astype(jnp.bfloat16), jnp.asarray(_R[1]).astype(jnp.bfloat16),
      jnp.asarray(_R[2]).astype(jnp.bfloat16), jnp.asarray(_R[3]).astype(jnp.bfloat16))

    return out_w, out_l, f_xlr


# speedup vs baseline: 1.0794x; 1.0794x over previous
import functools

import numpy as np
import jax
import jax.numpy as jnp
from jax import lax
from jax.experimental import pallas as pl
from jax.experimental.pallas import tpu as pltpu
from jax.experimental.pallas import tpu_sc as plsc

PATCH_BANK = [128, 256, 384, 512]
CS = 128
FOV_SCALE = 32
N_LOC = 8
H = W = 1024
B, C, K = 4, 3, 4
HL = H // FOV_SCALE


def _resize_mat(p, out):
    scale = out / p
    sample_f = (np.arange(out, dtype=np.float64) + 0.5) / scale - 0.5
    x = np.abs(sample_f[:, None] - np.arange(p, dtype=np.float64)[None, :]) * min(scale, 1.0)
    w = np.maximum(0.0, 1.0 - x)
    w = w / w.sum(axis=1, keepdims=True)
    return w.astype(np.float32)


_R = [_resize_mat(p, CS) for p in PATCH_BANK]
_D = _resize_mat(H, HL)


def _router_kernel(x_ref, d_ref, dt_ref, wf_ref, bf_ref, xi_ref, yi_ref,
                   f_ref, wloc_ref, x16_ref):
    x = x_ref[0]
    xlr = []
    for c in range(C):
        xb = x[c].astype(jnp.bfloat16)
        x16_ref[0, c] = xb
        a = jnp.dot(xb, dt_ref[...], preferred_element_type=jnp.float32)
        xlr.append(jnp.dot(d_ref[...], a.astype(jnp.bfloat16),
                           preferred_element_type=jnp.float32))
    pads = []
    zr = jnp.zeros((1, HL), jnp.float32)
    zc = jnp.zeros((HL + 2, 1), jnp.float32)
    for c in range(C):
        t = jnp.concatenate([zr, xlr[c], zr], axis=0)
        pads.append(jnp.concatenate([zc, t, zc], axis=1))
    logits = []
    for k in range(K):
        acc = jnp.full((HL, HL), 0.0, jnp.float32) + bf_ref[k, 0]
        for c in range(C):
            for dy in range(3):
                for dx in range(3):
                    acc = acc + wf_ref[k, c * 9 + dy * 3 + dx] * pads[c][dy:dy + HL, dx:dx + HL]
        logits.append(acc)
    m = jnp.maximum(jnp.maximum(logits[0], logits[1]),
                    jnp.maximum(logits[2], logits[3]))
    es = [jnp.exp(l - m) for l in logits]
    tot = es[0] + es[1] + es[2] + es[3]
    fs = [e / tot for e in es]
    for k in range(K):
        f_ref[0, k] = fs[k]
    ox = (jax.lax.broadcasted_iota(jnp.int32, (N_LOC, HL), 1) == xi_ref[...]).astype(jnp.float32)
    oy = (jax.lax.broadcasted_iota(jnp.int32, (N_LOC, HL), 1) == yi_ref[...]).astype(jnp.float32)
    cols = []
    for k in range(K):
        mk = jnp.dot(ox, fs[k], preferred_element_type=jnp.float32)
        cols.append(jnp.sum(mk * oy, axis=1, keepdims=True))
    wloc_ref[0] = jnp.concatenate(cols, axis=1)


WIN = [p + CS for p in PATCH_BANK]


_SC_NC, _SC_NS = 2, 16
_SC_NW = _SC_NC * _SC_NS
_LROWS = B * N_LOC * CS * (CS // 32)
_LPW = _LROWS // _SC_NW


def _label_sc_kernel(y32_hbm, idx_hbm, rl_hbm, out_hbm,
                     idx_v, buf2_v, out_v, rl_v, sem):
    wid = lax.axis_index("s") * _SC_NC + lax.axis_index("c")
    pltpu.sync_copy(idx_hbm.at[pl.ds(wid * 2 * CS, 2 * CS)], idx_v)
    pltpu.sync_copy(rl_hbm.at[wid], rl_v)
    pltpu.async_copy(y32_hbm.at[idx_v], buf2_v, sem).wait()
    masks = [rl_v[...] == 32 * q for q in range(4)]

    @pl.loop(0, CS)
    def _(r):
        for j in range(CS // 16):
            cands = []
            for q in range(4):
                col = 16 * j + 32 * q
                if col < 128:
                    cands.append(buf2_v[2 * r, pl.ds(col, 16)])
                else:
                    cands.append(buf2_v[2 * r + 1, pl.ds(col - 128, 16)])
            g = jnp.where(masks[0], cands[0],
                          jnp.where(masks[1], cands[1],
                                    jnp.where(masks[2], cands[2], cands[3])))
            out_v[r, pl.ds(16 * j, 16)] = g

    pltpu.sync_copy(out_v, out_hbm.at[pl.ds(wid * CS, CS)])


def _gather_labels(Y, xls, yls):
    y32 = Y.reshape(-1, 128)
    r = jnp.arange(CS, dtype=jnp.int32)
    b = jnp.arange(B, dtype=jnp.int32)
    cb = yls // 128
    rl = yls - cb * 128
    rowa = (b[None, :, None] * (H * W // 128)
            + (xls[:, None, None] + r[None, None, :]) * (W // 128)
            + cb[:, None, None])
    rowb = jnp.minimum(rowa + 1, H * W * B // 128 - 1)
    gidx = jnp.stack([rowa, rowb], axis=-1).astype(jnp.int32).reshape(-1)
    rl16 = jnp.broadcast_to(jnp.repeat(rl, B)[:, None].astype(jnp.int32),
                            (_SC_NW, 16))
    mesh = plsc.VectorSubcoreMesh(core_axis_name="c", subcore_axis_name="s")
    out = pl.kernel(
        _label_sc_kernel,
        out_type=jax.ShapeDtypeStruct((B * N_LOC * CS, CS), jnp.int32),
        mesh=mesh,
        scratch_types=[
            pltpu.VMEM((2 * CS,), jnp.int32),
            pltpu.VMEM((2 * CS, CS), jnp.int32),
            pltpu.VMEM((CS, CS), jnp.int32),
            pltpu.VMEM((16,), jnp.int32),
            pltpu.SemaphoreType.DMA,
        ],
    )(y32, gidx, rl16)
    return out.reshape(B * N_LOC, CS, CS)


def _patch_kernel(idx_ref, x_ref, wloc_ref,
                  g0_ref, g1_ref, g2_ref, g3_ref,
                  r0_ref, r1_ref, r2_ref, r3_ref,
                  out_w_ref, s_ref, sem_s):
    i = pl.program_id(0)

    def s_copy(j, slot):
        x0 = pl.multiple_of(idx_ref[j, 0], 32)
        y0a = pl.multiple_of(idx_ref[j, 1], 128)
        return pltpu.make_async_copy(
            x_ref.at[:, pl.ds(x0, 512), pl.ds(y0a, 640)],
            s_ref.at[slot], sem_s.at[slot])

    slot = jax.lax.rem(i, 2)
    nslot = jax.lax.rem(i + 1, 2)

    @pl.when(i == 0)
    def _():
        s_copy(i, slot).start()

    @pl.when(i + 1 < N_LOC)
    def _():
        s_copy(i + 1, nslot).start()

    s_copy(i, slot).wait()

    rrefs = [r0_ref, r1_ref, r2_ref, r3_ref]
    grefs = [g0_ref, g1_ref, g2_ref, g3_ref]
    for bc in range(B * C):
        b, c = bc // C, bc % C
        acc = jnp.zeros((CS, CS), jnp.float32)
        for k in range(K):
            p = PATCH_BANK[k]
            dx = pl.multiple_of(idx_ref[i, 2 + k], 32)
            base = pl.multiple_of(idx_ref[i, 6 + k], 128)
            srow = s_ref[slot, bc, pl.ds(dx, p), pl.ds(base, WIN[k])]
            t = jnp.dot(rrefs[k][...], srow,
                        preferred_element_type=jnp.float32)
            outk = jnp.dot(t.astype(jnp.bfloat16), grefs[k][0],
                           preferred_element_type=jnp.float32)
            acc = acc + wloc_ref[0, b, k] * outk
        out_w_ref[b, c] = acc


def kernel(X, Y, xi, yi, W_fov, b_fov):
    d = jnp.asarray(_D).astype(jnp.bfloat16)
    dt = jnp.asarray(_D.T).astype(jnp.bfloat16)
    wf = W_fov.reshape(K, C * 9)
    bf = b_fov.reshape(K, 1)
    xi2 = xi.reshape(N_LOC, 1)
    yi2 = yi.reshape(N_LOC, 1)

    router_outs = pl.pallas_call(
        _router_kernel,
        grid=(B,),
        in_specs=[
            pl.BlockSpec((1, C, H, W), lambda b: (b, 0, 0, 0)),
            pl.BlockSpec((HL, H), lambda b: (0, 0)),
            pl.BlockSpec((H, HL), lambda b: (0, 0)),
            pl.BlockSpec((K, C * 9), lambda b: (0, 0)),
            pl.BlockSpec((K, 1), lambda b: (0, 0)),
            pl.BlockSpec((N_LOC, 1), lambda b: (0, 0)),
            pl.BlockSpec((N_LOC, 1), lambda b: (0, 0)),
        ],
        out_specs=[
            pl.BlockSpec((1, K, HL, HL), lambda b: (b, 0, 0, 0)),
            pl.BlockSpec((1, N_LOC, K), lambda b: (b, 0, 0)),
            pl.BlockSpec((1, C, H, W), lambda b: (b, 0, 0, 0)),
        ],
        out_shape=[
            jax.ShapeDtypeStruct((B, K, HL, HL), jnp.float32),
            jax.ShapeDtypeStruct((B, N_LOC, K), jnp.float32),
            jax.ShapeDtypeStruct((B, C, H, W), jnp.bfloat16),
        ],
    )(X, d, dt, wf, bf, xi2, yi2)
    f_xlr, wloc_bk, x16 = router_outs

    wloc = jnp.transpose(wloc_bk, (1, 0, 2))

    cx = xi.astype(jnp.int32) * FOV_SCALE
    cy = yi.astype(jnp.int32) * FOV_SCALE
    x0s = jnp.clip(cx - 256, 0, H - 512)
    y0s = jnp.clip(cy - 256, 0, W - 512)
    xls = jnp.clip(cx - CS // 2, 0, H - CS)
    yls = jnp.clip(cy - CS // 2, 0, W - CS)
    y0a = jnp.minimum((y0s // 128) * 128, W - 640)
    ry = y0s - y0a
    dxs = [jnp.clip(cx - p // 2, 0, H - p) - x0s for p in PATCH_BANK]
    dys = [jnp.clip(cy - p // 2, 0, W - p) - y0s + ry for p in PATCH_BANK]
    bases = [jnp.minimum((dys[k] // 128) * 128, 640 - WIN[k]) for k in range(K)]
    offs = [dys[k] - bases[k] for k in range(K)]
    idx = jnp.stack([x0s, y0a] + dxs + bases, axis=1)

    gcts = []
    for k in range(K):
        rkt = jnp.asarray(_R[k].T)
        zero = jnp.zeros((WIN[k], CS), jnp.float32)
        gcts.append(jax.vmap(
            lambda off, r=rkt, z=zero: jax.lax.dynamic_update_slice(z, r, (off, 0))
        )(offs[k]).astype(jnp.bfloat16))

    out_l = _gather_labels(Y, xls, yls)

    x12 = x16.reshape(B * C, H, W)

    grid_spec = pltpu.PrefetchScalarGridSpec(
        num_scalar_prefetch=1,
        grid=(N_LOC,),
        in_specs=[
            pl.BlockSpec(memory_space=pl.ANY),
            pl.BlockSpec((1, B, K), lambda i, idx: (i, 0, 0)),
            pl.BlockSpec((1, WIN[0], CS), lambda i, idx: (i, 0, 0)),
            pl.BlockSpec((1, WIN[1], CS), lambda i, idx: (i, 0, 0)),
            pl.BlockSpec((1, WIN[2], CS), lambda i, idx: (i, 0, 0)),
            pl.BlockSpec((1, WIN[3], CS), lambda i, idx: (i, 0, 0)),
            pl.BlockSpec((CS, 128), lambda i, idx: (0, 0)),
            pl.BlockSpec((CS, 256), lambda i, idx: (0, 0)),
            pl.BlockSpec((CS, 384), lambda i, idx: (0, 0)),
            pl.BlockSpec((CS, 512), lambda i, idx: (0, 0)),
        ],
        out_specs=[
            pl.BlockSpec((B, C, CS, CS), lambda i, idx: (i, 0, 0, 0)),
        ],
        scratch_shapes=[
            pltpu.VMEM((2, B * C, 512, 640), jnp.bfloat16),
            pltpu.SemaphoreType.DMA((2,)),
        ],
    )

    out_w, = pl.pallas_call(
        _patch_kernel,
        grid_spec=grid_spec,
        out_shape=[
            jax.ShapeDtypeStruct((B * N_LOC, C, CS, CS), jnp.float32),
        ],
    )(idx, x12, wloc, gcts[0], gcts[1], gcts[2], gcts[3],
      jnp.asarray(_R[0]).astype(jnp.bfloat16), jnp.asarray(_R[1]).astype(jnp.bfloat16),
      jnp.asarray(_R[2]).astype(jnp.bfloat16), jnp.asarray(_R[3]).astype(jnp.bfloat16))

    return out_w, out_l, f_xlr
